# Initial kernel scaffold; baseline (speedup 1.0000x reference)
import jax
import jax.numpy as jnp
from jax.experimental import pallas as pl


def kernel(fea, edge_index, W1, b1, W2, b2, Wd1, bd1, Wd2, bd2):
    raise NotImplementedError("write your pallas kernel here")


# speedup vs baseline: 7.2213x; 7.2213x over previous
import functools

import jax
import jax.numpy as jnp
from jax import lax
from jax.experimental import pallas as pl
from jax.experimental.pallas import tpu as pltpu
from jax.experimental.pallas import tpu_sc as plsc

_N = 10000
_NPAD = 10240
_E = 320000
_NTILES = 32
_K = 125
_EPT = _E // _NTILES
_CHUNKS = _EPT // _K
_RPT = _NPAD // 16
_RDEC = 400


def _mm1_body(x_ref, w_ref, b_ref, o_ref):
    o_ref[...] = (
        jnp.dot(x_ref[...], w_ref[...], preferred_element_type=jnp.float32)
        + b_ref[...]
    )


def _mm1(x, w, b):
    n, din = x.shape
    dout = w.shape[1]
    return pl.pallas_call(
        _mm1_body,
        out_shape=jax.ShapeDtypeStruct((n, dout), jnp.float32),
    )(x, w, b)


def _mm2_body(p_ref, w_ref, b_ref, o_ref):
    h = jnp.maximum(p_ref[0] + p_ref[1], 0.0)
    o_ref[...] = (
        jnp.dot(h, w_ref[...], preferred_element_type=jnp.float32) + b_ref[...]
    )


def _mm2(p, w, b):
    n = p.shape[1]
    dout = w.shape[1]
    return pl.pallas_call(
        _mm2_body,
        out_shape=jax.ShapeDtypeStruct((n, dout), jnp.float32),
    )(p, w, b)


def _dec_body(zp_ref, zpb_ref, wd1_ref, bd1_ref, wd2_ref, bd2_ref,
              recon_ref, adj_ref):
    zfull = (zp_ref[0] + zp_ref[1])[:_N]
    zblk = zpb_ref[0] + zpb_ref[1]
    d = jnp.maximum(
        jnp.dot(zblk, wd1_ref[...], preferred_element_type=jnp.float32)
        + bd1_ref[...], 0.0)
    recon_ref[...] = jnp.maximum(
        jnp.dot(d, wd2_ref[...], preferred_element_type=jnp.float32)
        + bd2_ref[...], 0.0)
    logits = lax.dot_general(zblk, zfull, (((1,), (1,)), ((), ())),
                             preferred_element_type=jnp.float32)
    adj_ref[...] = jax.nn.sigmoid(logits)


def _decode(zp, wd1, bd1, wd2, bd2):
    n = _N
    npad = zp.shape[1]
    dz = zp.shape[2]
    d1 = wd1.shape[1]
    d0 = wd2.shape[1]
    grid = (n // _RDEC,)
    return pl.pallas_call(
        _dec_body,
        grid=grid,
        in_specs=[
            pl.BlockSpec((2, npad, dz), lambda i: (0, 0, 0)),
            pl.BlockSpec((2, _RDEC, dz), lambda i: (0, i, 0)),
            pl.BlockSpec((dz, d1), lambda i: (0, 0)),
            pl.BlockSpec((1, d1), lambda i: (0, 0)),
            pl.BlockSpec((d1, d0), lambda i: (0, 0)),
            pl.BlockSpec((1, d0), lambda i: (0, 0)),
        ],
        out_specs=[
            pl.BlockSpec((_RDEC, d0), lambda i: (i, 0)),
            pl.BlockSpec((_RDEC, n), lambda i: (i, 0)),
        ],
        out_shape=[
            jax.ShapeDtypeStruct((n, d0), jnp.float32),
            jax.ShapeDtypeStruct((n, n), jnp.float32),
        ],
    )(zp, zp, wd1, bd1, wd2, bd2)


def _make_segsum(d):
    mesh = plsc.VectorSubcoreMesh(core_axis_name="c", subcore_axis_name="s")

    @functools.partial(
        pl.kernel,
        out_type=jax.ShapeDtypeStruct((2, _NPAD, d), jnp.float32),
        mesh=mesh,
        scratch_types=[
            pltpu.VMEM((_CHUNKS, _K), jnp.int32),
            pltpu.VMEM((_CHUNKS, _K), jnp.int32),
            pltpu.VMEM((_K, d), jnp.float32),
            pltpu.VMEM_SHARED((_NPAD, d), jnp.float32),
            pltpu.SemaphoreType.DMA,
        ],
        compiler_params=pltpu.CompilerParams(use_tc_tiling_on_sc=False),
    )
    def segsum(sup_hbm, src_hbm, dst_hbm, zero_hbm, out_hbm,
               src_v, dst_v, rows_v, acc_sh, sem):
        cid = lax.axis_index("c")
        sid = lax.axis_index("s")
        tile = cid * 16 + sid
        r0 = sid * _RPT
        pltpu.sync_copy(zero_hbm.at[pl.ds(r0, _RPT)],
                        acc_sh.at[pl.ds(r0, _RPT)])
        c0 = tile * _CHUNKS
        pltpu.sync_copy(src_hbm.at[pl.ds(c0, _CHUNKS)], src_v)
        pltpu.sync_copy(dst_hbm.at[pl.ds(c0, _CHUNKS)], dst_v)
        plsc.subcore_barrier()

        def body(i, carry):
            pltpu.async_copy(sup_hbm.at[src_v.at[i]], rows_v, sem).wait()
            pltpu.sync_copy(rows_v, acc_sh.at[dst_v.at[i]], add=True)
            return carry

        lax.fori_loop(0, _CHUNKS, body, 0)
        plsc.subcore_barrier()
        pltpu.sync_copy(acc_sh.at[pl.ds(r0, _RPT)],
                        out_hbm.at[cid, pl.ds(r0, _RPT)])

    return segsum


_segsum64 = _make_segsum(64)
_segsum32 = _make_segsum(32)


def kernel(fea, edge_index, W1, b1, W2, b2, Wd1, bd1, Wd2, bd2):
    src = edge_index[0].reshape(_NTILES * _CHUNKS, _K)
    dst = edge_index[1].reshape(_NTILES * _CHUNKS, _K)
    zero64 = jnp.zeros((_NPAD, 64), jnp.float32)
    zero32 = jnp.zeros((_NPAD, 32), jnp.float32)

    sup1 = _mm1(fea, W1, b1.reshape(1, -1))
    p1 = _segsum64(sup1, src, dst, zero64)
    sup2 = _mm2(p1, W2, b2.reshape(1, -1))
    p2 = _segsum32(sup2, src, dst, zero32)
    recon, adj = _decode(p2, Wd1, bd1.reshape(1, -1), Wd2, bd2.reshape(1, -1))
    return recon, adj


# speedup vs baseline: 9.0375x; 1.2515x over previous
import functools

import jax
import jax.numpy as jnp
from jax import lax
from jax.experimental import pallas as pl
from jax.experimental.pallas import tpu as pltpu
from jax.experimental.pallas import tpu_sc as plsc

_N = 10000
_NPAD = 10240
_E = 320000
_NTILES = 32
_K = 125
_EPT = _E // _NTILES
_CHUNKS = _EPT // _K
_RPT = _NPAD // 16
_RDEC = 400


def _mm1_body(x_ref, w_ref, b_ref, o_ref):
    o_ref[...] = (
        jnp.dot(x_ref[...], w_ref[...], preferred_element_type=jnp.float32)
        + b_ref[...]
    )


def _mm1(x, w, b):
    n, din = x.shape
    dout = w.shape[1]
    return pl.pallas_call(
        _mm1_body,
        out_shape=jax.ShapeDtypeStruct((n, dout), jnp.float32),
    )(x, w, b)


def _mm2_body(p_ref, w_ref, b_ref, o_ref):
    h = jnp.maximum(p_ref[0] + p_ref[1], 0.0)
    o_ref[...] = (
        jnp.dot(h, w_ref[...], preferred_element_type=jnp.float32) + b_ref[...]
    )


def _mm2(p, w, b):
    n = p.shape[1]
    dout = w.shape[1]
    return pl.pallas_call(
        _mm2_body,
        out_shape=jax.ShapeDtypeStruct((n, dout), jnp.float32),
    )(p, w, b)


def _dec_body(zp_ref, zpb_ref, wd1_ref, bd1_ref, wd2_ref, bd2_ref,
              recon_ref, adj_ref):
    zfull = (zp_ref[0] + zp_ref[1])[:_N]
    zblk = zpb_ref[0] + zpb_ref[1]
    d = jnp.maximum(
        jnp.dot(zblk, wd1_ref[...], preferred_element_type=jnp.float32)
        + bd1_ref[...], 0.0)
    recon_ref[...] = jnp.maximum(
        jnp.dot(d, wd2_ref[...], preferred_element_type=jnp.float32)
        + bd2_ref[...], 0.0)
    logits = lax.dot_general(zblk, zfull, (((1,), (1,)), ((), ())),
                             preferred_element_type=jnp.float32)
    adj_ref[...] = jax.nn.sigmoid(logits)


def _decode(zp, wd1, bd1, wd2, bd2):
    n = _N
    npad = zp.shape[1]
    dz = zp.shape[2]
    d1 = wd1.shape[1]
    d0 = wd2.shape[1]
    grid = (n // _RDEC,)
    return pl.pallas_call(
        _dec_body,
        grid=grid,
        in_specs=[
            pl.BlockSpec((2, npad, dz), lambda i: (0, 0, 0)),
            pl.BlockSpec((2, _RDEC, dz), lambda i: (0, i, 0)),
            pl.BlockSpec((dz, d1), lambda i: (0, 0)),
            pl.BlockSpec((1, d1), lambda i: (0, 0)),
            pl.BlockSpec((d1, d0), lambda i: (0, 0)),
            pl.BlockSpec((1, d0), lambda i: (0, 0)),
        ],
        out_specs=[
            pl.BlockSpec((_RDEC, d0), lambda i: (i, 0)),
            pl.BlockSpec((_RDEC, n), lambda i: (i, 0)),
        ],
        out_shape=[
            jax.ShapeDtypeStruct((n, d0), jnp.float32),
            jax.ShapeDtypeStruct((n, n), jnp.float32),
        ],
    )(zp, zp, wd1, bd1, wd2, bd2)


def _make_segsum(d):
    mesh = plsc.VectorSubcoreMesh(core_axis_name="c", subcore_axis_name="s")

    @functools.partial(
        pl.kernel,
        out_type=jax.ShapeDtypeStruct((2, _NPAD, d), jnp.float32),
        mesh=mesh,
        scratch_types=[
            pltpu.VMEM((_CHUNKS, _K), jnp.int32),
            pltpu.VMEM((_CHUNKS, _K), jnp.int32),
            pltpu.VMEM((_K, d), jnp.float32),
            pltpu.VMEM((_K, d), jnp.float32),
            pltpu.VMEM_SHARED((_NPAD, d), jnp.float32),
            pltpu.SemaphoreType.DMA,
            pltpu.SemaphoreType.DMA,
        ],
        compiler_params=pltpu.CompilerParams(use_tc_tiling_on_sc=False),
    )
    def segsum(sup_hbm, src_hbm, dst_hbm, zero_hbm, out_hbm,
               src_v, dst_v, rows_v0, rows_v1, acc_sh, sem0, sem1):
        cid = lax.axis_index("c")
        sid = lax.axis_index("s")
        tile = cid * 16 + sid
        r0 = sid * _RPT
        pltpu.sync_copy(zero_hbm.at[pl.ds(r0, _RPT)],
                        acc_sh.at[pl.ds(r0, _RPT)])
        c0 = tile * _CHUNKS
        pltpu.sync_copy(src_hbm.at[pl.ds(c0, _CHUNKS)], src_v)
        pltpu.sync_copy(dst_hbm.at[pl.ds(c0, _CHUNKS)], dst_v)
        plsc.subcore_barrier()

        rows = (rows_v0, rows_v1)
        sems = (sem0, sem1)
        nb = 2
        for b in range(nb):
            pltpu.async_copy(sup_hbm.at[src_v.at[b]], rows[b], sems[b])

        def body(g, carry):
            for b in range(nb):
                i = g * nb + b
                pltpu.make_async_copy(sup_hbm.at[src_v.at[i]],
                                      rows[b], sems[b]).wait()
                pltpu.sync_copy(rows[b], acc_sh.at[dst_v.at[i]], add=True)
                nxt = i + nb

                @pl.when(nxt < _CHUNKS)
                def _():
                    pltpu.async_copy(sup_hbm.at[src_v.at[nxt]],
                                     rows[b], sems[b])
            return carry

        lax.fori_loop(0, _CHUNKS // nb, body, 0)
        plsc.subcore_barrier()
        pltpu.sync_copy(acc_sh.at[pl.ds(r0, _RPT)],
                        out_hbm.at[cid, pl.ds(r0, _RPT)])

    return segsum


_segsum64 = _make_segsum(64)
_segsum32 = _make_segsum(32)


def kernel(fea, edge_index, W1, b1, W2, b2, Wd1, bd1, Wd2, bd2):
    src = edge_index[0].reshape(_NTILES * _CHUNKS, _K)
    dst = edge_index[1].reshape(_NTILES * _CHUNKS, _K)
    zero64 = jnp.zeros((_NPAD, 64), jnp.float32)
    zero32 = jnp.zeros((_NPAD, 32), jnp.float32)

    sup1 = _mm1(fea, W1, b1.reshape(1, -1))
    p1 = _segsum64(sup1, src, dst, zero64)
    sup2 = _mm2(p1, W2, b2.reshape(1, -1))
    p2 = _segsum32(sup2, src, dst, zero32)
    recon, adj = _decode(p2, Wd1, bd1.reshape(1, -1), Wd2, bd2.reshape(1, -1))
    return recon, adj


# speedup vs baseline: 10.4774x; 1.1593x over previous
import functools

import jax
import jax.numpy as jnp
from jax import lax
from jax.experimental import pallas as pl
from jax.experimental.pallas import tpu as pltpu
from jax.experimental.pallas import tpu_sc as plsc

_N = 10000
_NPAD = 10240
_E = 320000
_NTILES = 32
_K = 125
_EPT = _E // _NTILES
_CHUNKS = _EPT // _K
_RPT = _NPAD // 16
_RDEC = 400


def _mm1_body(x_ref, w_ref, b_ref, o_ref):
    o_ref[...] = (
        jnp.dot(x_ref[...], w_ref[...], preferred_element_type=jnp.float32)
        + b_ref[...]
    )


def _mm1(x, w, b):
    n, din = x.shape
    dout = w.shape[1]
    return pl.pallas_call(
        _mm1_body,
        out_shape=jax.ShapeDtypeStruct((n, dout), jnp.float32),
    )(x, w, b)


def _mm2_body(p_ref, w_ref, b_ref, o_ref):
    h = jnp.maximum(p_ref[0] + p_ref[1], 0.0)
    o_ref[...] = (
        jnp.dot(h, w_ref[...], preferred_element_type=jnp.float32) + b_ref[...]
    )


def _mm2(p, w, b):
    n = p.shape[1]
    dout = w.shape[1]
    return pl.pallas_call(
        _mm2_body,
        out_shape=jax.ShapeDtypeStruct((n, dout), jnp.float32),
    )(p, w, b)


def _dec_body(zp_ref, zpb_ref, wd1_ref, bd1_ref, wd2_ref, bd2_ref,
              recon_ref, adj_ref):
    zfull = (zp_ref[0] + zp_ref[1])[:_N]
    zblk = zpb_ref[0] + zpb_ref[1]
    d = jnp.maximum(
        jnp.dot(zblk, wd1_ref[...], preferred_element_type=jnp.float32)
        + bd1_ref[...], 0.0)
    recon_ref[...] = jnp.maximum(
        jnp.dot(d, wd2_ref[...], preferred_element_type=jnp.float32)
        + bd2_ref[...], 0.0)
    logits = lax.dot_general(zblk, zfull, (((1,), (1,)), ((), ())),
                             preferred_element_type=jnp.float32)
    adj_ref[...] = 0.5 * jnp.tanh(0.5 * logits) + 0.5


def _decode(zp, wd1, bd1, wd2, bd2):
    n = _N
    npad = zp.shape[1]
    dz = zp.shape[2]
    d1 = wd1.shape[1]
    d0 = wd2.shape[1]
    grid = (n // _RDEC,)
    return pl.pallas_call(
        _dec_body,
        grid=grid,
        in_specs=[
            pl.BlockSpec((2, npad, dz), lambda i: (0, 0, 0)),
            pl.BlockSpec((2, _RDEC, dz), lambda i: (0, i, 0)),
            pl.BlockSpec((dz, d1), lambda i: (0, 0)),
            pl.BlockSpec((1, d1), lambda i: (0, 0)),
            pl.BlockSpec((d1, d0), lambda i: (0, 0)),
            pl.BlockSpec((1, d0), lambda i: (0, 0)),
        ],
        out_specs=[
            pl.BlockSpec((_RDEC, d0), lambda i: (i, 0)),
            pl.BlockSpec((_RDEC, n), lambda i: (i, 0)),
        ],
        out_shape=[
            jax.ShapeDtypeStruct((n, d0), jnp.float32),
            jax.ShapeDtypeStruct((n, n), jnp.float32),
        ],
    )(zp, zp, wd1, bd1, wd2, bd2)


def _make_segsum(d):
    mesh = plsc.VectorSubcoreMesh(core_axis_name="c", subcore_axis_name="s")

    @functools.partial(
        pl.kernel,
        out_type=jax.ShapeDtypeStruct((2, _NPAD, d), jnp.float32),
        mesh=mesh,
        scratch_types=[
            pltpu.VMEM((_CHUNKS, _K), jnp.int32),
            pltpu.VMEM((_CHUNKS, _K), jnp.int32),
            pltpu.VMEM((_K, d), jnp.float32),
            pltpu.VMEM((_K, d), jnp.float32),
            pltpu.VMEM((_K, d), jnp.float32),
            pltpu.VMEM((_K, d), jnp.float32),
            pltpu.VMEM_SHARED((_NPAD, d), jnp.float32),
            pltpu.SemaphoreType.DMA,
            pltpu.SemaphoreType.DMA,
            pltpu.SemaphoreType.DMA,
            pltpu.SemaphoreType.DMA,
        ],
        compiler_params=pltpu.CompilerParams(use_tc_tiling_on_sc=False),
    )
    def segsum(sup_hbm, src_hbm, dst_hbm, zero_hbm, out_hbm,
               src_v, dst_v, rows_v0, rows_v1, rows_v2, rows_v3, acc_sh,
               sem0, sem1, sem2, sem3):
        cid = lax.axis_index("c")
        sid = lax.axis_index("s")
        tile = cid * 16 + sid
        r0 = sid * _RPT
        pltpu.sync_copy(zero_hbm.at[pl.ds(r0, _RPT)],
                        acc_sh.at[pl.ds(r0, _RPT)])
        c0 = tile * _CHUNKS
        pltpu.sync_copy(src_hbm.at[pl.ds(c0, _CHUNKS)], src_v)
        pltpu.sync_copy(dst_hbm.at[pl.ds(c0, _CHUNKS)], dst_v)
        plsc.subcore_barrier()

        rows = (rows_v0, rows_v1, rows_v2, rows_v3)
        sems = (sem0, sem1, sem2, sem3)
        nb = 4
        for b in range(nb):
            pltpu.async_copy(sup_hbm.at[src_v.at[b]], rows[b], sems[b])

        def body(g, carry):
            for b in range(nb):
                i = g * nb + b
                pltpu.make_async_copy(sup_hbm.at[src_v.at[i]],
                                      rows[b], sems[b]).wait()
                pltpu.sync_copy(rows[b], acc_sh.at[dst_v.at[i]], add=True)
                nxt = i + nb

                @pl.when(nxt < _CHUNKS)
                def _():
                    pltpu.async_copy(sup_hbm.at[src_v.at[nxt]],
                                     rows[b], sems[b])
            return carry

        lax.fori_loop(0, _CHUNKS // nb, body, 0)
        plsc.subcore_barrier()
        pltpu.sync_copy(acc_sh.at[pl.ds(r0, _RPT)],
                        out_hbm.at[cid, pl.ds(r0, _RPT)])

    return segsum


_segsum64 = _make_segsum(64)
_segsum32 = _make_segsum(32)


def kernel(fea, edge_index, W1, b1, W2, b2, Wd1, bd1, Wd2, bd2):
    src = edge_index[0].reshape(_NTILES * _CHUNKS, _K)
    dst = edge_index[1].reshape(_NTILES * _CHUNKS, _K)
    zero64 = jnp.zeros((_NPAD, 64), jnp.float32)
    zero32 = jnp.zeros((_NPAD, 32), jnp.float32)

    sup1 = _mm1(fea, W1, b1.reshape(1, -1))
    p1 = _segsum64(sup1, src, dst, zero64)
    sup2 = _mm2(p1, W2, b2.reshape(1, -1))
    p2 = _segsum32(sup2, src, dst, zero32)
    recon, adj = _decode(p2, Wd1, bd1.reshape(1, -1), Wd2, bd2.reshape(1, -1))
    return recon, adj


# speedup vs baseline: 10.5283x; 1.0049x over previous
import functools

import jax
import jax.numpy as jnp
from jax import lax
from jax.experimental import pallas as pl
from jax.experimental.pallas import tpu as pltpu
from jax.experimental.pallas import tpu_sc as plsc

_N = 10000
_NPAD = 10240
_E = 320000
_NTILES = 32
_K = 125
_EPT = _E // _NTILES
_CHUNKS = _EPT // _K
_RPT = _NPAD // 16
_RDEC = 400
_NB = 8


def _mm1_body(x_ref, w_ref, b_ref, o_ref):
    o_ref[...] = (
        jnp.dot(x_ref[...], w_ref[...], preferred_element_type=jnp.float32)
        + b_ref[...]
    )


def _mm1(x, w, b):
    n, din = x.shape
    dout = w.shape[1]
    return pl.pallas_call(
        _mm1_body,
        out_shape=jax.ShapeDtypeStruct((n, dout), jnp.float32),
    )(x, w, b)


def _mm2_body(p_ref, w_ref, b_ref, o_ref):
    h = jnp.maximum(p_ref[0] + p_ref[1], 0.0)
    o_ref[...] = (
        jnp.dot(h, w_ref[...], preferred_element_type=jnp.float32) + b_ref[...]
    )


def _mm2(p, w, b):
    n = p.shape[1]
    dout = w.shape[1]
    return pl.pallas_call(
        _mm2_body,
        out_shape=jax.ShapeDtypeStruct((n, dout), jnp.float32),
    )(p, w, b)


def _dec_body(zp_ref, zpb_ref, wd1_ref, bd1_ref, wd2_ref, bd2_ref,
              recon_ref, adj_ref):
    zfull = (zp_ref[0] + zp_ref[1])[:_N]
    zblk = zpb_ref[0] + zpb_ref[1]
    d = jnp.maximum(
        jnp.dot(zblk, wd1_ref[...], preferred_element_type=jnp.float32)
        + bd1_ref[...], 0.0)
    recon_ref[...] = jnp.maximum(
        jnp.dot(d, wd2_ref[...], preferred_element_type=jnp.float32)
        + bd2_ref[...], 0.0)
    logits = lax.dot_general(zblk, zfull, (((1,), (1,)), ((), ())),
                             preferred_element_type=jnp.float32)
    adj_ref[...] = 0.5 * jnp.tanh(0.5 * logits) + 0.5


def _decode(zp, wd1, bd1, wd2, bd2):
    n = _N
    npad = zp.shape[1]
    dz = zp.shape[2]
    d1 = wd1.shape[1]
    d0 = wd2.shape[1]
    grid = (n // _RDEC,)
    return pl.pallas_call(
        _dec_body,
        grid=grid,
        in_specs=[
            pl.BlockSpec((2, npad, dz), lambda i: (0, 0, 0)),
            pl.BlockSpec((2, _RDEC, dz), lambda i: (0, i, 0)),
            pl.BlockSpec((dz, d1), lambda i: (0, 0)),
            pl.BlockSpec((1, d1), lambda i: (0, 0)),
            pl.BlockSpec((d1, d0), lambda i: (0, 0)),
            pl.BlockSpec((1, d0), lambda i: (0, 0)),
        ],
        out_specs=[
            pl.BlockSpec((_RDEC, d0), lambda i: (i, 0)),
            pl.BlockSpec((_RDEC, n), lambda i: (i, 0)),
        ],
        out_shape=[
            jax.ShapeDtypeStruct((n, d0), jnp.float32),
            jax.ShapeDtypeStruct((n, n), jnp.float32),
        ],
    )(zp, zp, wd1, bd1, wd2, bd2)


def _make_segsum(d):
    mesh = plsc.VectorSubcoreMesh(core_axis_name="c", subcore_axis_name="s")

    @functools.partial(
        pl.kernel,
        out_type=jax.ShapeDtypeStruct((2, _NPAD, d), jnp.float32),
        mesh=mesh,
        scratch_types=[
            pltpu.VMEM((_CHUNKS, _K), jnp.int32),
            pltpu.VMEM((_CHUNKS, _K), jnp.int32),
          ] + [pltpu.VMEM((_K, d), jnp.float32)] * _NB
          + [pltpu.VMEM_SHARED((_NPAD, d), jnp.float32)]
          + [pltpu.SemaphoreType.DMA] * _NB,
        compiler_params=pltpu.CompilerParams(use_tc_tiling_on_sc=False),
    )
    def segsum(sup_hbm, src_hbm, dst_hbm, zero_hbm, out_hbm,
               src_v, dst_v, *bufs):
        rows = bufs[:_NB]
        acc_sh = bufs[_NB]
        sems = bufs[_NB + 1:]
        cid = lax.axis_index("c")
        sid = lax.axis_index("s")
        tile = cid * 16 + sid
        r0 = sid * _RPT
        pltpu.sync_copy(zero_hbm.at[pl.ds(r0, _RPT)],
                        acc_sh.at[pl.ds(r0, _RPT)])
        c0 = tile * _CHUNKS
        pltpu.sync_copy(src_hbm.at[pl.ds(c0, _CHUNKS)], src_v)
        pltpu.sync_copy(dst_hbm.at[pl.ds(c0, _CHUNKS)], dst_v)
        plsc.subcore_barrier()

        nb = _NB
        for b in range(nb):
            pltpu.async_copy(sup_hbm.at[src_v.at[b]], rows[b], sems[b])

        def body(g, carry):
            for b in range(nb):
                i = g * nb + b
                pltpu.make_async_copy(sup_hbm.at[src_v.at[i]],
                                      rows[b], sems[b]).wait()
                pltpu.sync_copy(rows[b], acc_sh.at[dst_v.at[i]], add=True)
                nxt = i + nb

                @pl.when(nxt < _CHUNKS)
                def _():
                    pltpu.async_copy(sup_hbm.at[src_v.at[nxt]],
                                     rows[b], sems[b])
            return carry

        lax.fori_loop(0, _CHUNKS // nb, body, 0)
        plsc.subcore_barrier()
        pltpu.sync_copy(acc_sh.at[pl.ds(r0, _RPT)],
                        out_hbm.at[cid, pl.ds(r0, _RPT)])

    return segsum


_segsum64 = _make_segsum(64)
_segsum32 = _make_segsum(32)


def kernel(fea, edge_index, W1, b1, W2, b2, Wd1, bd1, Wd2, bd2):
    src = edge_index[0].reshape(_NTILES * _CHUNKS, _K)
    dst = edge_index[1].reshape(_NTILES * _CHUNKS, _K)
    zero64 = jnp.zeros((_NPAD, 64), jnp.float32)
    zero32 = jnp.zeros((_NPAD, 32), jnp.float32)

    sup1 = _mm1(fea, W1, b1.reshape(1, -1))
    p1 = _segsum64(sup1, src, dst, zero64)
    sup2 = _mm2(p1, W2, b2.reshape(1, -1))
    p2 = _segsum32(sup2, src, dst, zero32)
    recon, adj = _decode(p2, Wd1, bd1.reshape(1, -1), Wd2, bd2.reshape(1, -1))
    return recon, adj
